# baseline (speedup 1.0000x reference)
import numpy as np
import jax
import jax.numpy as jnp
from jax import lax
from jax.experimental import pallas as pl
from jax.experimental.pallas import tpu as pltpu
from jax.experimental.pallas import tpu_sc as plsc

VOCAB = 100000
SEED = 42
PRIOR = 4
NTILE = 32
SC_SPAN = 25000
CHUNK = 800
NVEC = CHUNK // 16
LAST_BASE = SC_SPAN - CHUNK
TC_SPAN = VOCAB - SC_SPAN

_ROT_A = (13, 15, 26, 6)
_ROT_B = (17, 29, 16, 24)


def _np_threefry(k0, k1, x0, x1):
    k0 = np.uint32(k0); k1 = np.uint32(k1)
    ks2 = np.uint32(k0 ^ k1 ^ np.uint32(0x1BD11BDA))
    ks = [k0, k1, ks2]
    x0 = (np.asarray([x0], np.uint32) + k0).astype(np.uint32)
    x1 = (np.asarray([x1], np.uint32) + k1).astype(np.uint32)
    for d in range(5):
        for r in (_ROT_A if d % 2 == 0 else _ROT_B):
            x0 = (x0 + x1).astype(np.uint32)
            x1 = ((x1 << np.uint32(r)) | (x1 >> np.uint32(32 - r))).astype(np.uint32)
            x1 = (x1 ^ x0).astype(np.uint32)
        x0 = (x0 + ks[(d + 1) % 3]).astype(np.uint32)
        x1 = (x1 + ks[(d + 2) % 3] + np.uint32(d + 1)).astype(np.uint32)
    return x0[0], x1[0]

_FK0, _FK1 = _np_threefry(0, SEED, 0, 0)
_FK0 = int(_FK0); _FK1 = int(_FK1)


def _rotl(x, r):
    return (x << jnp.uint32(r)) | (x >> jnp.uint32(32 - r))


def _key_schedule(k0, k1):
    ks2 = k0 ^ k1 ^ jnp.uint32(0x1BD11BDA)
    ks = (k0, k1, ks2)
    inj = [(k0, k1)]
    for d in range(5):
        inj.append((ks[(d + 1) % 3], ks[(d + 2) % 3] + jnp.uint32(d + 1)))
    return inj


def _cipher(inj, x0, x1):
    x0 = x0 + inj[0][0]
    x1 = x1 + inj[0][1]
    for d in range(5):
        for r in (_ROT_A if d % 2 == 0 else _ROT_B):
            x0 = x0 + x1
            x1 = _rotl(x1, r)
            x1 = x1 ^ x0
        x0 = x0 + inj[d + 1][0]
        x1 = x1 + inj[d + 1][1]
    return x0, x1


_SQRT2 = 1.4142135
_LN2 = 0.6931471805599453


def _flog(x):
    bits = lax.bitcast_convert_type(x, jnp.uint32)
    e = (bits >> jnp.uint32(23)).astype(jnp.int32) - 127
    m = lax.bitcast_convert_type(
        (bits & jnp.uint32(0x7FFFFF)) | jnp.uint32(0x3F800000), jnp.float32)
    big = m >= _SQRT2
    m = jnp.where(big, m * 0.5, m)
    e = e + jnp.where(big, 1, 0)
    z = (m - 1.0) / (m + 1.0)
    z2 = z * z
    p = 0.22222222
    for c in (0.28571429, 0.4, 0.66666667, 2.0):
        p = p * z2 + c
    return e.astype(jnp.float32) * _LN2 + z * p


def _score_kernel(ids_hbm, logits_hbm, part_hbm, ids_v, log_v,
                  val_v, idx_v):
    nc = 2
    wid = lax.axis_index("s") * nc + lax.axis_index("c")
    base = pl.multiple_of(jnp.where(wid == NTILE - 1, LAST_BASE, wid * CHUNK), 8)
    pltpu.sync_copy(ids_hbm.at[0, pl.ds(2048 - 16, 16)], ids_v)
    pltpu.sync_copy(logits_hbm.at[pl.ds(base, CHUNK)], log_v)

    iota = lax.iota(jnp.int32, 16)
    ids = ids_v[...]
    prior = jnp.sum(jnp.where(iota >= 16 - PRIOR, ids, 0))

    pv = jnp.broadcast_to(prior.astype(jnp.uint32), (16,))
    zero_v = jnp.zeros((16,), jnp.uint32)
    inj0 = _key_schedule(jnp.uint32(_FK0), jnp.uint32(_FK1))
    k0v, k1v = _cipher(inj0, zero_v, pv)
    inj = _key_schedule(k0v, k1v)

    big_s = jnp.full((16,), 3e38, jnp.float32)
    UNROLL = 2

    def body(j, carry):
        vmin, vidx = carry
        for q in range(UNROLL):
            gbase = base + (j * UNROLL + q) * 16
            idxv = gbase + iota
            x1 = idxv.astype(jnp.uint32)
            o0, o1 = _cipher(inj, zero_v, x1)
            ubits = o0 ^ o1
            u = lax.bitcast_convert_type(
                (ubits >> jnp.uint32(9)) | jnp.uint32(0x3F800000),
                jnp.float32) - 1.0
            logu = _flog(jnp.where(u == 0.0, 1.0, u))
            s = _flog(0.0 - logu) - log_v[pl.ds((j * UNROLL + q) * 16, 16)]
            s = jnp.where(u == 0.0, big_s, s)
            upd = s < vmin
            vmin = jnp.where(upd, s, vmin)
            vidx = jnp.where(upd, idxv, vidx)
        return vmin, vidx

    vmin, vidx = lax.fori_loop(0, NVEC // UNROLL, body,
                               (big_s, jnp.zeros((16,), jnp.int32)))

    m = jnp.min(vmin)
    cand = jnp.where(vmin == m, vidx, jnp.int32(2 ** 30))
    mi = jnp.min(cand)
    val_v[...] = jnp.broadcast_to(m, (16,))
    idx_v[...] = jnp.broadcast_to(mi.astype(jnp.float32), (16,))
    pltpu.sync_copy(val_v, part_hbm.at[pl.ds(wid * 16, 16)])
    pltpu.sync_copy(idx_v, part_hbm.at[pl.ds(512 + wid * 16, 16)])


_TC_BLK = 25000
_TC_G = TC_SPAN // _TC_BLK


def _tc_score_kernel(ids_ref, logits_ref, out_ref, m_s, i_s):
    i = pl.program_id(0)
    ids = ids_ref[0, pl.ds(2048 - PRIOR, PRIOR)]
    prior = jnp.sum(ids)
    inj0 = _key_schedule(jnp.uint32(_FK0), jnp.uint32(_FK1))
    k0, k1 = _cipher(inj0, jnp.uint32(0), prior.astype(jnp.uint32))
    inj = _key_schedule(k0, k1)

    shape = (1, 2, 12500)
    col = ((i + 1) * _TC_BLK
           + lax.broadcasted_iota(jnp.int32, shape, 1) * 12500
           + lax.broadcasted_iota(jnp.int32, shape, 2))
    o0, o1 = _cipher(inj, jnp.zeros(shape, jnp.uint32), col.astype(jnp.uint32))
    u = lax.bitcast_convert_type(
        ((o0 ^ o1) >> jnp.uint32(9)) | jnp.uint32(0x3F800000),
        jnp.float32) - 1.0
    s = jnp.log(0.0 - jnp.log(u)) - logits_ref[...]
    m = jnp.min(s)
    mi = jnp.min(jnp.where(s == m, col, jnp.int32(2 ** 30)))

    @pl.when((i == 0) | (m < m_s[0]))
    def _():
        m_s[0] = m
        i_s[0] = mi

    @pl.when(i == _TC_G - 1)
    def _():
        lane = lax.broadcasted_iota(jnp.int32, (1, 128), 1)
        out_ref[...] = jnp.where(
            lane == 0, m_s[0],
            jnp.where(lane == 1, i_s[0].astype(jnp.float32), 0.0))


def _emit_kernel(part_ref, tc_ref, out_ref):
    vals = part_ref[pl.ds(0, 512)]
    idxs = part_ref[pl.ds(512, 512)]
    m = jnp.min(vals)
    gidx = jnp.min(jnp.where(vals == m, idxs, 3e38)).astype(jnp.int32)
    tc_m = tc_ref[0, 0]
    tc_i = tc_ref[0, 1].astype(jnp.int32)
    gidx = jnp.where(tc_m < m, tc_i, gidx)
    col = lax.broadcasted_iota(jnp.int32, (1, VOCAB), 1)
    out_ref[...] = jnp.where(col == gidx, 100000.0, -100000.0)


def kernel(input_ids, logits):
    mesh = plsc.VectorSubcoreMesh(core_axis_name="c", subcore_axis_name="s")

    score = pl.kernel(
        _score_kernel,
        out_type=jax.ShapeDtypeStruct((NTILE * 32,), jnp.float32),
        mesh=mesh,
        scratch_types=[
            pltpu.VMEM((16,), jnp.int32),
            pltpu.VMEM((CHUNK,), jnp.float32),
            pltpu.VMEM((16,), jnp.float32),
            pltpu.VMEM((16,), jnp.float32),
        ],
        compiler_params=pltpu.CompilerParams(
            needs_layout_passes=False, use_tc_tiling_on_sc=False),
    )
    logits_flat = logits.reshape(VOCAB)
    partials = score(input_ids, logits_flat)

    logits3d = logits.reshape(4, 2, 12500)
    tc_part = pl.pallas_call(
        _tc_score_kernel,
        out_shape=jax.ShapeDtypeStruct((1, 128), jnp.float32),
        grid=(_TC_G,),
        in_specs=[pl.BlockSpec((1, 2048), lambda i: (0, 0)),
                  pl.BlockSpec((1, 2, 12500), lambda i: (i + 1, 0, 0))],
        out_specs=pl.BlockSpec((1, 128), lambda i: (0, 0)),
        scratch_shapes=[pltpu.SMEM((1,), jnp.float32),
                        pltpu.SMEM((1,), jnp.int32)],
    )(input_ids, logits3d)

    out = pl.pallas_call(
        _emit_kernel,
        out_shape=jax.ShapeDtypeStruct((1, VOCAB), jnp.float32),
    )(partials, tc_part)
    return out


# speedup vs baseline: 1.2963x; 1.2963x over previous
import numpy as np
import jax
import jax.numpy as jnp
from jax import lax
from jax.experimental import pallas as pl
from jax.experimental.pallas import tpu as pltpu
from jax.experimental.pallas import tpu_sc as plsc

VOCAB = 100000
SEED = 42
PRIOR = 4
NTILE = 32
SC_SPAN = 25000
CHUNK = 800
NVEC = CHUNK // 16
LAST_BASE = SC_SPAN - CHUNK
TC_SPAN = VOCAB - SC_SPAN

_ROT_A = (13, 15, 26, 6)
_ROT_B = (17, 29, 16, 24)


def _np_threefry(k0, k1, x0, x1):
    k0 = np.uint32(k0); k1 = np.uint32(k1)
    ks2 = np.uint32(k0 ^ k1 ^ np.uint32(0x1BD11BDA))
    ks = [k0, k1, ks2]
    x0 = (np.asarray([x0], np.uint32) + k0).astype(np.uint32)
    x1 = (np.asarray([x1], np.uint32) + k1).astype(np.uint32)
    for d in range(5):
        for r in (_ROT_A if d % 2 == 0 else _ROT_B):
            x0 = (x0 + x1).astype(np.uint32)
            x1 = ((x1 << np.uint32(r)) | (x1 >> np.uint32(32 - r))).astype(np.uint32)
            x1 = (x1 ^ x0).astype(np.uint32)
        x0 = (x0 + ks[(d + 1) % 3]).astype(np.uint32)
        x1 = (x1 + ks[(d + 2) % 3] + np.uint32(d + 1)).astype(np.uint32)
    return x0[0], x1[0]

_FK0, _FK1 = _np_threefry(0, SEED, 0, 0)
_FK0 = int(_FK0); _FK1 = int(_FK1)


def _rotl(x, r):
    return (x << jnp.uint32(r)) | (x >> jnp.uint32(32 - r))


def _key_schedule(k0, k1):
    ks2 = k0 ^ k1 ^ jnp.uint32(0x1BD11BDA)
    ks = (k0, k1, ks2)
    inj = [(k0, k1)]
    for d in range(5):
        inj.append((ks[(d + 1) % 3], ks[(d + 2) % 3] + jnp.uint32(d + 1)))
    return inj


def _cipher(inj, x0, x1):
    x0 = x0 + inj[0][0]
    x1 = x1 + inj[0][1]
    for d in range(5):
        for r in (_ROT_A if d % 2 == 0 else _ROT_B):
            x0 = x0 + x1
            x1 = _rotl(x1, r)
            x1 = x1 ^ x0
        x0 = x0 + inj[d + 1][0]
        x1 = x1 + inj[d + 1][1]
    return x0, x1


_SQRT2 = 1.4142135
_LN2 = 0.6931471805599453


def _flog(x):
    bits = lax.bitcast_convert_type(x, jnp.uint32)
    e = (bits >> jnp.uint32(23)).astype(jnp.int32) - 127
    m = lax.bitcast_convert_type(
        (bits & jnp.uint32(0x7FFFFF)) | jnp.uint32(0x3F800000), jnp.float32)
    big = m >= _SQRT2
    m = jnp.where(big, m * 0.5, m)
    e = e + jnp.where(big, 1, 0)
    z = (m - 1.0) / (m + 1.0)
    z2 = z * z
    p = 0.22222222
    for c in (0.28571429, 0.4, 0.66666667, 2.0):
        p = p * z2 + c
    return e.astype(jnp.float32) * _LN2 + z * p


def _score_kernel(ids_hbm, logits_hbm, part_hbm, ids_v, log_v,
                  val_v, idx_v):
    nc = 2
    wid = lax.axis_index("s") * nc + lax.axis_index("c")
    base = pl.multiple_of(jnp.where(wid == NTILE - 1, LAST_BASE, wid * CHUNK), 8)
    pltpu.sync_copy(ids_hbm.at[0, pl.ds(2048 - 16, 16)], ids_v)
    pltpu.sync_copy(logits_hbm.at[pl.ds(base, CHUNK)], log_v)

    iota = lax.iota(jnp.int32, 16)
    ids = ids_v[...]
    prior = jnp.sum(jnp.where(iota >= 16 - PRIOR, ids, 0))

    pv = jnp.broadcast_to(prior.astype(jnp.uint32), (16,))
    zero_v = jnp.zeros((16,), jnp.uint32)
    inj0 = _key_schedule(jnp.uint32(_FK0), jnp.uint32(_FK1))
    k0v, k1v = _cipher(inj0, zero_v, pv)
    inj = _key_schedule(k0v, k1v)

    big_s = jnp.full((16,), 3e38, jnp.float32)
    UNROLL = 2

    def body(j, carry):
        vmin, vidx = carry
        for q in range(UNROLL):
            gbase = base + (j * UNROLL + q) * 16
            idxv = gbase + iota
            x1 = idxv.astype(jnp.uint32)
            o0, o1 = _cipher(inj, zero_v, x1)
            ubits = o0 ^ o1
            u = lax.bitcast_convert_type(
                (ubits >> jnp.uint32(9)) | jnp.uint32(0x3F800000),
                jnp.float32) - 1.0
            logu = _flog(jnp.where(u == 0.0, 1.0, u))
            s = _flog(0.0 - logu) - log_v[pl.ds((j * UNROLL + q) * 16, 16)]
            s = jnp.where(u == 0.0, big_s, s)
            upd = s < vmin
            vmin = jnp.where(upd, s, vmin)
            vidx = jnp.where(upd, idxv, vidx)
        return vmin, vidx

    vmin, vidx = lax.fori_loop(0, NVEC // UNROLL, body,
                               (big_s, jnp.zeros((16,), jnp.int32)))

    m = jnp.min(vmin)
    cand = jnp.where(vmin == m, vidx, jnp.int32(2 ** 30))
    mi = jnp.min(cand)
    val_v[...] = jnp.broadcast_to(m, (16,))
    idx_v[...] = jnp.broadcast_to(mi.astype(jnp.float32), (16,))
    pltpu.sync_copy(val_v, part_hbm.at[pl.ds(wid * 16, 16)])
    pltpu.sync_copy(idx_v, part_hbm.at[pl.ds(512 + wid * 16, 16)])


def _tc_score_kernel(ids_ref, logits_ref, out_ref):
    ids = ids_ref[0, pl.ds(2048 - PRIOR, PRIOR)]
    prior = jnp.sum(ids)
    inj0 = _key_schedule(jnp.uint32(_FK0), jnp.uint32(_FK1))
    k0, k1 = _cipher(inj0, jnp.uint32(0), prior.astype(jnp.uint32))
    inj = _key_schedule(k0, k1)

    shape = (3, 8, 3125)
    col = (SC_SPAN
           + lax.broadcasted_iota(jnp.int32, shape, 0) * 25000
           + lax.broadcasted_iota(jnp.int32, shape, 1) * 3125
           + lax.broadcasted_iota(jnp.int32, shape, 2))
    o0, o1 = _cipher(inj, jnp.zeros(shape, jnp.uint32), col.astype(jnp.uint32))
    u = lax.bitcast_convert_type(
        ((o0 ^ o1) >> jnp.uint32(9)) | jnp.uint32(0x3F800000),
        jnp.float32) - 1.0
    s = jnp.log(0.0 - jnp.log(u)) - logits_ref[...]
    m = jnp.min(s)
    mi = jnp.min(jnp.where(s == m, col, jnp.int32(2 ** 30)))
    lane = lax.broadcasted_iota(jnp.int32, (1, 128), 1)
    out_ref[...] = jnp.where(lane == 0, m,
                             jnp.where(lane == 1, mi.astype(jnp.float32), 0.0))


def _emit_kernel(part_ref, tc_ref, out_ref):
    vals = part_ref[pl.ds(0, 512)]
    idxs = part_ref[pl.ds(512, 512)]
    m = jnp.min(vals)
    gidx = jnp.min(jnp.where(vals == m, idxs, 3e38)).astype(jnp.int32)
    tc_m = tc_ref[0, 0]
    tc_i = tc_ref[0, 1].astype(jnp.int32)
    gidx = jnp.where(tc_m < m, tc_i, gidx)
    col = lax.broadcasted_iota(jnp.int32, (1, VOCAB), 1)
    out_ref[...] = jnp.where(col == gidx, 100000.0, -100000.0)


def kernel(input_ids, logits):
    mesh = plsc.VectorSubcoreMesh(core_axis_name="c", subcore_axis_name="s")

    score = pl.kernel(
        _score_kernel,
        out_type=jax.ShapeDtypeStruct((NTILE * 32,), jnp.float32),
        mesh=mesh,
        scratch_types=[
            pltpu.VMEM((16,), jnp.int32),
            pltpu.VMEM((CHUNK,), jnp.float32),
            pltpu.VMEM((16,), jnp.float32),
            pltpu.VMEM((16,), jnp.float32),
        ],
        compiler_params=pltpu.CompilerParams(
            needs_layout_passes=False, use_tc_tiling_on_sc=False),
    )
    logits_sc = logits[0, :SC_SPAN]
    partials = score(input_ids, logits_sc)

    logits4d = logits.reshape(4, 8, 3125)
    tc_part = pl.pallas_call(
        _tc_score_kernel,
        out_shape=jax.ShapeDtypeStruct((1, 128), jnp.float32),
        grid=(1,),
        in_specs=[pl.BlockSpec((1, 2048), lambda i: (0, 0)),
                  pl.BlockSpec((3, 8, 3125), lambda i: (1, 0, 0))],
        out_specs=pl.BlockSpec((1, 128), lambda i: (0, 0)),
    )(input_ids, logits4d)

    out = pl.pallas_call(
        _emit_kernel,
        out_shape=jax.ShapeDtypeStruct((1, VOCAB), jnp.float32),
    )(partials, tc_part)
    return out
